# baseline (speedup 1.0000x reference)
import functools

import jax
import jax.numpy as jnp
from jax import lax
from jax.experimental import pallas as pl
from jax.experimental.pallas import tpu as pltpu
from jax.experimental.pallas import tpu_sc as plsc

N = 10000
NPAD = 10240
M = 2500
MPAD = 2560
K = 32
E = MPAD * K
DIM_IN = 128
DIM = DIM_IN + 4
DP = 256
DIM_OUT = 128
R2 = 0.25
BIGF = 1048576.0

NW = 32
EPW = E // NW
CHUNKS = EPW // 128

_f32 = jnp.float32
_i32 = jnp.int32


def _fps_body(px, py, pz, pack, idx_o, qx_o, qy_o, qz_o, nx_o, ny_o, nz_o):
    fio = (lax.broadcasted_iota(_i32, (80, 128), 0) * 128
           + lax.broadcasted_iota(_i32, (80, 128), 1)).astype(_f32)
    mio = (lax.broadcasted_iota(_i32, (20, 128), 0) * 128
           + lax.broadcasted_iota(_i32, (20, 128), 1))
    pxv = px[...]
    pyv = py[...]
    pzv = pz[...]
    dists0 = jnp.where(fio < float(N), jnp.inf, -jnp.inf).astype(_f32)

    row0 = pack[pl.ds(0, 1), :]
    zacc = jnp.zeros((20, 128), _f32)
    m0 = mio == 0
    accI0 = zacc
    accX0 = jnp.where(m0, row0[0, 0], zacc)
    accY0 = jnp.where(m0, row0[0, 1], zacc)
    accZ0 = jnp.where(m0, row0[0, 2], zacc)
    accNX0 = jnp.where(m0, row0[0, 3], zacc)
    accNY0 = jnp.where(m0, row0[0, 4], zacc)
    accNZ0 = jnp.where(m0, row0[0, 5], zacc)

    def body(i, c):
        (dists, lx, ly, lz, accI, accX, accY, accZ, accNX, accNY, accNZ) = c
        dx = pxv - lx
        dy = pyv - ly
        dz = pzv - lz
        d = dx * dx + dy * dy + dz * dz
        dists = jnp.minimum(dists, d)
        mval = jnp.max(dists)
        cand = jnp.where(dists == mval, fio, BIGF)
        nxtf = jnp.min(cand)
        nxt = nxtf.astype(_i32)
        row = pack[pl.ds(nxt, 1), :]
        mi = mio == i
        accI = jnp.where(mi, nxtf, accI)
        accX = jnp.where(mi, row[0, 0], accX)
        accY = jnp.where(mi, row[0, 1], accY)
        accZ = jnp.where(mi, row[0, 2], accZ)
        accNX = jnp.where(mi, row[0, 3], accNX)
        accNY = jnp.where(mi, row[0, 4], accNY)
        accNZ = jnp.where(mi, row[0, 5], accNZ)
        return (dists, row[0, 0], row[0, 1], row[0, 2],
                accI, accX, accY, accZ, accNX, accNY, accNZ)

    c = (dists0, row0[0, 0], row0[0, 1], row0[0, 2],
         accI0, accX0, accY0, accZ0, accNX0, accNY0, accNZ0)
    c = lax.fori_loop(1, M, body, c)
    (_, _, _, _, accI, accX, accY, accZ, accNX, accNY, accNZ) = c
    idx_o[...] = accI.astype(_i32)
    qx_o[...] = accX
    qy_o[...] = accY
    qz_o[...] = accZ
    nx_o[...] = accNX
    ny_o[...] = accNY
    nz_o[...] = accNZ


def _fps(px, py, pz, pack):
    outs = ([jax.ShapeDtypeStruct((20, 128), _i32)]
            + [jax.ShapeDtypeStruct((20, 128), _f32)] * 6)
    return pl.pallas_call(_fps_body, out_shape=outs)(px, py, pz, pack)


def _radius_body(q8, pT, cols_o, valid_o):
    q = q8[...]
    p = pT[...]
    ddx = q[:, 0:1] - p[0:1, :]
    ddy = q[:, 1:2] - p[1:2, :]
    ddz = q[:, 2:3] - p[2:3, :]
    d2 = ddx * ddx + ddy * ddy + ddz * ddz
    colf = lax.broadcasted_iota(_i32, (64, NPAD), 1).astype(_f32)
    keyv = jnp.where(d2 <= R2, colf, BIGF)
    lio = lax.broadcasted_iota(_i32, (64, 128), 1)
    acc0 = jnp.full((64, 128), BIGF, _f32)

    def body(t, c):
        keyv, acc = c
        mv = jnp.min(keyv, axis=1, keepdims=True)
        acc = jnp.where(lio == t, mv, acc)
        keyv = jnp.where(keyv == mv, BIGF, keyv)
        return keyv, acc

    _, acc = lax.fori_loop(0, K, body, (keyv, acc0))
    ok = acc < float(N)
    cols_o[...] = jnp.where(ok, acc, 0.0).astype(_i32)
    valid_o[...] = ok.astype(_f32)


def _radius(q8, pT):
    grid = MPAD // 64
    return pl.pallas_call(
        _radius_body,
        grid=(grid,),
        in_specs=[
            pl.BlockSpec((64, 8), lambda i: (i, 0)),
            pl.BlockSpec((8, NPAD), lambda i: (0, 0)),
        ],
        out_specs=[
            pl.BlockSpec((64, 128), lambda i: (i, 0)),
            pl.BlockSpec((64, 128), lambda i: (i, 0)),
        ],
        out_shape=[
            jax.ShapeDtypeStruct((MPAD, 128), _i32),
            jax.ShapeDtypeStruct((MPAD, 128), _f32),
        ],
    )(q8, pT)


def _sc_gather(table, idx3):
    mesh = plsc.VectorSubcoreMesh(core_axis_name="c", subcore_axis_name="s")

    NB = 3

    @functools.partial(
        pl.kernel, mesh=mesh,
        out_type=jax.ShapeDtypeStruct((E, DP), _f32),
        scratch_types=[
            pltpu.VMEM((CHUNKS, 128), _i32),
            pltpu.VMEM((NB, 128, DP), _f32),
            pltpu.SemaphoreType.DMA((NB,)),
            pltpu.SemaphoreType.DMA((NB,)),
        ],
    )
    def kfn(t_hbm, idx_hbm, out_hbm, idx_v, bufs, gsem, wsem):
        wid = lax.axis_index("s") * 2 + lax.axis_index("c")
        pltpu.sync_copy(idx_hbm.at[wid], idx_v)
        base = wid * EPW

        g = {}
        w = {}
        for j in range(NB):
            g[j] = pltpu.async_copy(t_hbm.at[idx_v.at[j]], bufs.at[j],
                                    gsem.at[j])
        for j in range(CHUNKS):
            b = j % NB
            g[j].wait()
            w[j] = pltpu.async_copy(bufs.at[b],
                                    out_hbm.at[pl.ds(base + j * 128, 128)],
                                    wsem.at[b])
            jn = j + NB
            if jn < CHUNKS:
                w[j].wait()
                g[jn] = pltpu.async_copy(t_hbm.at[idx_v.at[jn]], bufs.at[b],
                                         gsem.at[b])
            else:
                w[j].wait()

    return kfn(table, idx3)


def _edge_body(gath, qgeoT, vm, w1x, w1p, b1, w2, b2, out_o):
    blk = gath[...]
    g = jnp.transpose(blk[:, DIM_IN:DIM_IN + 8])
    q = qgeoT[...]
    dx = g[0:1] - q[0:1]
    dy = g[1:2] - q[1:2]
    dz = g[2:3] - q[2:3]
    nix, niy, niz = q[3:4], q[4:5], q[5:6]
    njx, njy, njz = g[3:4], g[4:5], g[5:6]

    r0 = jnp.sqrt(dx * dx + dy * dy + dz * dz + 1e-12)

    def ang(ax, ay, az, bx, by, bz):
        cx = ay * bz - az * by
        cy = az * bx - ax * bz
        cz = ax * by - ay * bx
        cn = jnp.sqrt(cx * cx + cy * cy + cz * cz + 1e-12)
        dt = ax * bx + ay * by + az * bz
        return jnp.arctan2(cn, dt)

    a1 = ang(nix, niy, niz, dx, dy, dz)
    a2 = ang(njx, njy, njz, dx, dy, dz)
    a3 = ang(nix, niy, niz, njx, njy, njz)
    ppfT = jnp.concatenate([r0, a1, a2, a3, jnp.zeros((4, 1024), _f32)], axis=0)

    h1 = jnp.dot(blk[:, :DIM_IN], w1x[...], preferred_element_type=_f32)
    h1 = h1 + lax.dot_general(ppfT, w1p[...], (((0,), (0,)), ((), ())),
                              preferred_element_type=_f32)
    h1 = jnp.maximum(h1 + b1[...], 0.0)
    h2 = jnp.maximum(jnp.dot(h1, w2[...], preferred_element_type=_f32)
                     + b2[...], 0.0)

    qi = lax.broadcasted_iota(_i32, (32, 1024), 0)
    ei = lax.broadcasted_iota(_i32, (32, 1024), 1) // K
    sel = jnp.where(qi == ei, vm[...], 0.0)
    out_o[...] = jnp.dot(sel, h2, preferred_element_type=_f32)


def _edge_mlp(gath, qgeoT, vmrow, w1x, w1p, b1, w2, b2):
    grid = E // 1024
    return pl.pallas_call(
        _edge_body,
        grid=(grid,),
        in_specs=[
            pl.BlockSpec((1024, DP), lambda i: (i, 0)),
            pl.BlockSpec((8, 1024), lambda i: (0, i)),
            pl.BlockSpec((1, 1024), lambda i: (0, i)),
            pl.BlockSpec((DIM_IN, DP), lambda i: (0, 0)),
            pl.BlockSpec((8, DP), lambda i: (0, 0)),
            pl.BlockSpec((1, DP), lambda i: (0, 0)),
            pl.BlockSpec((DP, DP), lambda i: (0, 0)),
            pl.BlockSpec((1, DP), lambda i: (0, 0)),
        ],
        out_specs=pl.BlockSpec((32, DP), lambda i: (i, 0)),
        out_shape=jax.ShapeDtypeStruct((MPAD, DP), _f32),
    )(gath, qgeoT, vmrow, w1x, w1p, b1, w2, b2)


def _head_body(sums, valid, w3, b3, out_o):
    cnt = jnp.sum(valid[...], axis=1, keepdims=True)
    agg = sums[...] / jnp.maximum(cnt, 1.0)
    out = jnp.dot(agg, w3[...], preferred_element_type=_f32) + b3[...]
    out_o[...] = jnp.maximum(out, 0.0)


def _head(sums, valid, w3, b3):
    grid = MPAD // 256
    return pl.pallas_call(
        _head_body,
        grid=(grid,),
        in_specs=[
            pl.BlockSpec((256, DP), lambda i: (i, 0)),
            pl.BlockSpec((256, 128), lambda i: (i, 0)),
            pl.BlockSpec((DP, DIM_OUT), lambda i: (0, 0)),
            pl.BlockSpec((1, DIM_OUT), lambda i: (0, 0)),
        ],
        out_specs=pl.BlockSpec((256, DIM_OUT), lambda i: (i, 0)),
        out_shape=jax.ShapeDtypeStruct((MPAD, DIM_OUT), _f32),
    )(sums, valid, w3, b3)


def kernel(x, pos, batch, norm, W1, b1, W2, b2, W3, b3):
    padn = NPAD - N
    px = jnp.pad(pos[:, 0], (0, padn)).reshape(80, 128)
    py = jnp.pad(pos[:, 1], (0, padn)).reshape(80, 128)
    pz = jnp.pad(pos[:, 2], (0, padn)).reshape(80, 128)
    pack = jnp.pad(jnp.concatenate([pos, norm], axis=1),
                   ((0, padn), (0, 2)))
    pT = jnp.pad(pos.T, ((0, 5), (0, padn)), constant_values=1e3)
    pT = pT.at[3:, :].set(0.0).at[:3, N:].set(1e3)
    table = jnp.pad(jnp.concatenate([x, pos, norm], axis=1),
                    ((0, padn), (0, DP - DIM_IN - 6)))

    idx2d, qx, qy, qz, qnx, qny, qnz = _fps(px, py, pz, pack)

    q8 = jnp.stack([qx.reshape(-1), qy.reshape(-1), qz.reshape(-1)]
                   + [jnp.zeros((MPAD,), _f32)] * 5, axis=1)
    cols2d, valid2d = _radius(q8, pT)

    cols_e = cols2d[:, :K].reshape(E)
    idx3 = cols_e.reshape(NW, CHUNKS, 128)
    gath = _sc_gather(table, idx3)

    qgeo = jnp.stack([qx.reshape(-1), qy.reshape(-1), qz.reshape(-1),
                      qnx.reshape(-1), qny.reshape(-1), qnz.reshape(-1),
                      jnp.zeros((MPAD,), _f32), jnp.zeros((MPAD,), _f32)],
                     axis=0)
    qgeoT = jnp.repeat(qgeo, K, axis=1)
    vmrow = valid2d[:, :K].reshape(1, E)

    w1x = jnp.pad(W1[:DIM_IN, :], ((0, 0), (0, DP - DIM)))
    w1p = jnp.pad(W1[DIM_IN:, :], ((0, 4), (0, DP - DIM)))
    b1p = jnp.pad(b1, (0, DP - DIM)).reshape(1, DP)
    w2p = jnp.pad(W2, ((0, DP - DIM), (0, DP - DIM)))
    b2p = jnp.pad(b2, (0, DP - DIM)).reshape(1, DP)
    w3p = jnp.pad(W3, ((0, DP - DIM), (0, 0)))
    b3p = b3.reshape(1, DIM_OUT)

    sums = _edge_mlp(gath, qgeoT, vmrow, w1x, w1p, b1p, w2p, b2p)

    outp = _head(sums, valid2d, w3p, b3p)

    idx = idx2d.reshape(-1)[:M]
    out = outp[:M]
    qpos = jnp.stack([qx.reshape(-1)[:M], qy.reshape(-1)[:M],
                      qz.reshape(-1)[:M]], axis=1)
    qbatch = jnp.zeros((M,), _i32)
    return (out, qpos, qbatch, idx)


# speedup vs baseline: 1.0193x; 1.0193x over previous
import functools

import jax
import jax.numpy as jnp
from jax import lax
from jax.experimental import pallas as pl
from jax.experimental.pallas import tpu as pltpu
from jax.experimental.pallas import tpu_sc as plsc

N = 10000
NPAD = 10240
M = 2500
MPAD = 2560
K = 32
E = MPAD * K
DIM_IN = 128
DIM = DIM_IN + 4
DP = 256
TW = 128
DIM_OUT = 128
R2 = 0.25
BIGF = 1048576.0

NW = 32
EPW = E // NW
CHUNKS = EPW // 128

_f32 = jnp.float32
_i32 = jnp.int32


def _fps_body(px, py, pz, pack, idx_o, qx_o, qy_o, qz_o, nx_o, ny_o, nz_o):
    fio = (lax.broadcasted_iota(_i32, (80, 128), 0) * 128
           + lax.broadcasted_iota(_i32, (80, 128), 1)).astype(_f32)
    mio = (lax.broadcasted_iota(_i32, (20, 128), 0) * 128
           + lax.broadcasted_iota(_i32, (20, 128), 1))
    pxv = px[...]
    pyv = py[...]
    pzv = pz[...]
    dists0 = jnp.where(fio < float(N), jnp.inf, -jnp.inf).astype(_f32)

    row0 = pack[pl.ds(0, 1), :]
    zacc = jnp.zeros((20, 128), _f32)
    m0 = mio == 0
    accI0 = zacc
    accX0 = jnp.where(m0, row0[0, 0], zacc)
    accY0 = jnp.where(m0, row0[0, 1], zacc)
    accZ0 = jnp.where(m0, row0[0, 2], zacc)
    accNX0 = jnp.where(m0, row0[0, 3], zacc)
    accNY0 = jnp.where(m0, row0[0, 4], zacc)
    accNZ0 = jnp.where(m0, row0[0, 5], zacc)

    def body(i, c):
        (dists, lx, ly, lz, accI, accX, accY, accZ, accNX, accNY, accNZ) = c
        dx = pxv - lx
        dy = pyv - ly
        dz = pzv - lz
        d = dx * dx + dy * dy + dz * dz
        dists = jnp.minimum(dists, d)
        mval = jnp.max(dists)
        cand = jnp.where(dists == mval, fio, BIGF)
        nxtf = jnp.min(cand)
        nxt = nxtf.astype(_i32)
        row = pack[pl.ds(nxt, 1), :]
        mi = mio == i
        accI = jnp.where(mi, nxtf, accI)
        accX = jnp.where(mi, row[0, 0], accX)
        accY = jnp.where(mi, row[0, 1], accY)
        accZ = jnp.where(mi, row[0, 2], accZ)
        accNX = jnp.where(mi, row[0, 3], accNX)
        accNY = jnp.where(mi, row[0, 4], accNY)
        accNZ = jnp.where(mi, row[0, 5], accNZ)
        return (dists, row[0, 0], row[0, 1], row[0, 2],
                accI, accX, accY, accZ, accNX, accNY, accNZ)

    c = (dists0, row0[0, 0], row0[0, 1], row0[0, 2],
         accI0, accX0, accY0, accZ0, accNX0, accNY0, accNZ0)
    c = lax.fori_loop(1, M, body, c)
    (_, _, _, _, accI, accX, accY, accZ, accNX, accNY, accNZ) = c
    idx_o[...] = accI.astype(_i32)
    qx_o[...] = accX
    qy_o[...] = accY
    qz_o[...] = accZ
    nx_o[...] = accNX
    ny_o[...] = accNY
    nz_o[...] = accNZ


def _fps(px, py, pz, pack):
    outs = ([jax.ShapeDtypeStruct((20, 128), _i32)]
            + [jax.ShapeDtypeStruct((20, 128), _f32)] * 6)
    return pl.pallas_call(_fps_body, out_shape=outs)(px, py, pz, pack)


def _radius_body(q8, pT, cols_o, valid_o):
    q = q8[...]
    p = pT[...]
    ddx = q[:, 0:1] - p[0:1, :]
    ddy = q[:, 1:2] - p[1:2, :]
    ddz = q[:, 2:3] - p[2:3, :]
    d2 = ddx * ddx + ddy * ddy + ddz * ddz
    colf = lax.broadcasted_iota(_i32, (64, NPAD), 1).astype(_f32)
    keyv = jnp.where(d2 <= R2, colf, BIGF)
    lio = lax.broadcasted_iota(_i32, (64, 128), 1)
    acc0 = jnp.full((64, 128), BIGF, _f32)

    def body(t, c):
        keyv, acc = c
        mv = jnp.min(keyv, axis=1, keepdims=True)
        acc = jnp.where(lio == t, mv, acc)
        keyv = jnp.where(keyv == mv, BIGF, keyv)
        return keyv, acc

    _, acc = lax.fori_loop(0, K, body, (keyv, acc0))
    ok = acc < float(N)
    cols_o[...] = jnp.where(ok, acc, 0.0).astype(_i32)
    valid_o[...] = ok.astype(_f32)


def _radius(q8, pT):
    grid = MPAD // 64
    return pl.pallas_call(
        _radius_body,
        grid=(grid,),
        in_specs=[
            pl.BlockSpec((64, 8), lambda i: (i, 0)),
            pl.BlockSpec((8, NPAD), lambda i: (0, 0)),
        ],
        out_specs=[
            pl.BlockSpec((64, 128), lambda i: (i, 0)),
            pl.BlockSpec((64, 128), lambda i: (i, 0)),
        ],
        out_shape=[
            jax.ShapeDtypeStruct((MPAD, 128), _i32),
            jax.ShapeDtypeStruct((MPAD, 128), _f32),
        ],
    )(q8, pT)


def _sc_gather(table, idx3):
    mesh = plsc.VectorSubcoreMesh(core_axis_name="c", subcore_axis_name="s")

    NB = 3

    @functools.partial(
        pl.kernel, mesh=mesh,
        out_type=jax.ShapeDtypeStruct((E, TW), _f32),
        scratch_types=[
            pltpu.VMEM((CHUNKS, 128), _i32),
            pltpu.VMEM((NB, 128, TW), _f32),
            pltpu.SemaphoreType.DMA((NB,)),
            pltpu.SemaphoreType.DMA((NB,)),
        ],
    )
    def kfn(t_hbm, idx_hbm, out_hbm, idx_v, bufs, gsem, wsem):
        wid = lax.axis_index("s") * 2 + lax.axis_index("c")
        pltpu.sync_copy(idx_hbm.at[wid], idx_v)
        base = wid * EPW

        g = {}
        w = {}
        for j in range(NB):
            g[j] = pltpu.async_copy(t_hbm.at[idx_v.at[j]], bufs.at[j],
                                    gsem.at[j])
        for j in range(CHUNKS):
            b = j % NB
            g[j].wait()
            w[j] = pltpu.async_copy(bufs.at[b],
                                    out_hbm.at[pl.ds(base + j * 128, 128)],
                                    wsem.at[b])
            jn = j + NB
            if jn < CHUNKS:
                w[j].wait()
                g[jn] = pltpu.async_copy(t_hbm.at[idx_v.at[jn]], bufs.at[b],
                                         gsem.at[b])
            else:
                w[j].wait()

    return kfn(table, idx3)


def _edge_body(gath, qgeoT, vm, w1xe, w1xo, w1p, b1, w2, b2, out_o):
    blk = gath[...]
    u = lax.bitcast_convert_type(blk[:, :64], _i32)
    flo = lax.bitcast_convert_type(u << 16, _f32)
    fhi = lax.bitcast_convert_type(u & jnp.int32(-65536), _f32)
    g = jnp.transpose(blk[:, 64:72])
    q = qgeoT[...]
    dx = g[0:1] - q[0:1]
    dy = g[1:2] - q[1:2]
    dz = g[2:3] - q[2:3]
    nix, niy, niz = q[3:4], q[4:5], q[5:6]
    njx, njy, njz = g[3:4], g[4:5], g[5:6]

    r0 = jnp.sqrt(dx * dx + dy * dy + dz * dz + 1e-12)

    def ang(ax, ay, az, bx, by, bz):
        cx = ay * bz - az * by
        cy = az * bx - ax * bz
        cz = ax * by - ay * bx
        cn = jnp.sqrt(cx * cx + cy * cy + cz * cz + 1e-12)
        dt = ax * bx + ay * by + az * bz
        return jnp.arctan2(cn, dt)

    a1 = ang(nix, niy, niz, dx, dy, dz)
    a2 = ang(njx, njy, njz, dx, dy, dz)
    a3 = ang(nix, niy, niz, njx, njy, njz)
    ppfT = jnp.concatenate([r0, a1, a2, a3, jnp.zeros((4, 1024), _f32)], axis=0)

    h1 = (jnp.dot(flo, w1xe[...], preferred_element_type=_f32)
          + jnp.dot(fhi, w1xo[...], preferred_element_type=_f32))
    h1 = h1 + lax.dot_general(ppfT, w1p[...], (((0,), (0,)), ((), ())),
                              preferred_element_type=_f32)
    h1 = jnp.maximum(h1 + b1[...], 0.0)
    h2 = jnp.maximum(jnp.dot(h1, w2[...], preferred_element_type=_f32)
                     + b2[...], 0.0)

    qi = lax.broadcasted_iota(_i32, (32, 1024), 0)
    ei = lax.broadcasted_iota(_i32, (32, 1024), 1) // K
    sel = jnp.where(qi == ei, vm[...], 0.0)
    out_o[...] = jnp.dot(sel, h2, preferred_element_type=_f32)


def _edge_mlp(gath, qgeoT, vmrow, w1xe, w1xo, w1p, b1, w2, b2):
    grid = E // 1024
    return pl.pallas_call(
        _edge_body,
        grid=(grid,),
        in_specs=[
            pl.BlockSpec((1024, TW), lambda i: (i, 0)),
            pl.BlockSpec((8, 1024), lambda i: (0, i)),
            pl.BlockSpec((1, 1024), lambda i: (0, i)),
            pl.BlockSpec((64, DP), lambda i: (0, 0)),
            pl.BlockSpec((64, DP), lambda i: (0, 0)),
            pl.BlockSpec((8, DP), lambda i: (0, 0)),
            pl.BlockSpec((1, DP), lambda i: (0, 0)),
            pl.BlockSpec((DP, DP), lambda i: (0, 0)),
            pl.BlockSpec((1, DP), lambda i: (0, 0)),
        ],
        out_specs=pl.BlockSpec((32, DP), lambda i: (i, 0)),
        out_shape=jax.ShapeDtypeStruct((MPAD, DP), _f32),
    )(gath, qgeoT, vmrow, w1xe, w1xo, w1p, b1, w2, b2)


def _head_body(sums, valid, w3, b3, out_o):
    cnt = jnp.sum(valid[...], axis=1, keepdims=True)
    agg = sums[...] / jnp.maximum(cnt, 1.0)
    out = jnp.dot(agg, w3[...], preferred_element_type=_f32) + b3[...]
    out_o[...] = jnp.maximum(out, 0.0)


def _head(sums, valid, w3, b3):
    grid = MPAD // 256
    return pl.pallas_call(
        _head_body,
        grid=(grid,),
        in_specs=[
            pl.BlockSpec((256, DP), lambda i: (i, 0)),
            pl.BlockSpec((256, 128), lambda i: (i, 0)),
            pl.BlockSpec((DP, DIM_OUT), lambda i: (0, 0)),
            pl.BlockSpec((1, DIM_OUT), lambda i: (0, 0)),
        ],
        out_specs=pl.BlockSpec((256, DIM_OUT), lambda i: (i, 0)),
        out_shape=jax.ShapeDtypeStruct((MPAD, DIM_OUT), _f32),
    )(sums, valid, w3, b3)


def kernel(x, pos, batch, norm, W1, b1, W2, b2, W3, b3):
    padn = NPAD - N
    px = jnp.pad(pos[:, 0], (0, padn)).reshape(80, 128)
    py = jnp.pad(pos[:, 1], (0, padn)).reshape(80, 128)
    pz = jnp.pad(pos[:, 2], (0, padn)).reshape(80, 128)
    pack = jnp.pad(jnp.concatenate([pos, norm], axis=1),
                   ((0, padn), (0, 2)))
    pT = jnp.pad(pos.T, ((0, 5), (0, padn)), constant_values=1e3)
    pT = pT.at[3:, :].set(0.0).at[:3, N:].set(1e3)
    x16 = x.astype(jnp.bfloat16)
    xpk = lax.bitcast_convert_type(x16.reshape(N, 64, 2), _f32)
    table = jnp.pad(jnp.concatenate([xpk, pos, norm], axis=1),
                    ((0, padn), (0, TW - 70)))

    idx2d, qx, qy, qz, qnx, qny, qnz = _fps(px, py, pz, pack)

    q8 = jnp.stack([qx.reshape(-1), qy.reshape(-1), qz.reshape(-1)]
                   + [jnp.zeros((MPAD,), _f32)] * 5, axis=1)
    cols2d, valid2d = _radius(q8, pT)

    cols_e = cols2d[:, :K].reshape(E)
    idx3 = cols_e.reshape(NW, CHUNKS, 128)
    gath = _sc_gather(table, idx3)

    qgeo = jnp.stack([qx.reshape(-1), qy.reshape(-1), qz.reshape(-1),
                      qnx.reshape(-1), qny.reshape(-1), qnz.reshape(-1),
                      jnp.zeros((MPAD,), _f32), jnp.zeros((MPAD,), _f32)],
                     axis=0)
    qgeoT = jnp.repeat(qgeo, K, axis=1)
    vmrow = valid2d[:, :K].reshape(1, E)

    w1xe = jnp.pad(W1[0:DIM_IN:2, :], ((0, 0), (0, DP - DIM)))
    w1xo = jnp.pad(W1[1:DIM_IN:2, :], ((0, 0), (0, DP - DIM)))
    w1p = jnp.pad(W1[DIM_IN:, :], ((0, 4), (0, DP - DIM)))
    b1p = jnp.pad(b1, (0, DP - DIM)).reshape(1, DP)
    w2p = jnp.pad(W2, ((0, DP - DIM), (0, DP - DIM)))
    b2p = jnp.pad(b2, (0, DP - DIM)).reshape(1, DP)
    w3p = jnp.pad(W3, ((0, DP - DIM), (0, 0)))
    b3p = b3.reshape(1, DIM_OUT)

    sums = _edge_mlp(gath, qgeoT, vmrow, w1xe, w1xo, w1p, b1p, w2p, b2p)

    outp = _head(sums, valid2d, w3p, b3p)

    idx = idx2d.reshape(-1)[:M]
    out = outp[:M]
    qpos = jnp.stack([qx.reshape(-1)[:M], qy.reshape(-1)[:M],
                      qz.reshape(-1)[:M]], axis=1)
    qbatch = jnp.zeros((M,), _i32)
    return (out, qpos, qbatch, idx)
